# baseline (speedup 1.0000x reference)
import functools

import jax
import jax.numpy as jnp
from jax import lax
from jax.experimental import pallas as pl
from jax.experimental.pallas import tpu as pltpu
from jax.experimental.pallas import tpu_sc as plsc

E = 64
H = 1024
F = 512
T = 4096
CAP = 128
BT = 512
NB = T // BT
SENT = E * CAP
GRID_E = 66
SLOTS = GRID_E * CAP

NC, NS = 2, 16
NW = NC * NS


def _router_body(x_ref, rw_ref, slot_ref, prob_ref, counts_ref):
    b = pl.program_id(0)

    @pl.when(b == 0)
    def _():
        counts_ref[...] = jnp.zeros_like(counts_ref)

    x = x_ref[...]
    rw = rw_ref[...]
    logits = jnp.dot(x, rw, preferred_element_type=jnp.float32)
    m = jnp.max(logits, axis=1, keepdims=True)
    denom = jnp.sum(jnp.exp(logits - m), axis=1)
    prob = 1.0 / denom

    ids = lax.broadcasted_iota(jnp.int32, (BT, E), 1)
    is_max = logits == m
    expert = jnp.min(jnp.where(is_max, ids, E), axis=1)
    onehot = (ids == expert[:, None]).astype(jnp.float32)

    ti = lax.broadcasted_iota(jnp.int32, (BT, BT), 0)
    tj = lax.broadcasted_iota(jnp.int32, (BT, BT), 1)
    tri = (tj < ti).astype(jnp.float32)
    rank_mat = jnp.dot(tri, onehot, preferred_element_type=jnp.float32)
    rank = jnp.sum(rank_mat * onehot, axis=1)
    base = jnp.sum(counts_ref[...] * onehot, axis=1)
    pos = base + rank
    valid = pos < CAP
    slot = jnp.where(valid, expert * CAP + pos.astype(jnp.int32), SENT)
    counts_ref[...] = counts_ref[...] + jnp.sum(onehot, axis=0, keepdims=True)

    slot_ref[...] = slot.reshape(1, 1, BT)
    prob_ref[...] = prob.reshape(1, 1, BT)


def _router(hidden, rw):
    return pl.pallas_call(
        _router_body,
        grid=(NB,),
        in_specs=[
            pl.BlockSpec((BT, H), lambda b: (b, 0)),
            pl.BlockSpec((H, E), lambda b: (0, 0)),
        ],
        out_specs=[
            pl.BlockSpec((1, 1, BT), lambda b: (b, 0, 0)),
            pl.BlockSpec((1, 1, BT), lambda b: (b, 0, 0)),
        ],
        out_shape=[
            jax.ShapeDtypeStruct((NB, 1, BT), jnp.int32),
            jax.ShapeDtypeStruct((NB, 1, BT), jnp.float32),
        ],
        scratch_shapes=[pltpu.VMEM((1, E), jnp.float32)],
    )(hidden, rw)


def _mesh():
    return plsc.VectorSubcoreMesh(
        core_axis_name="c", subcore_axis_name="s", num_cores=NC, num_subcores=NS
    )


@functools.cache
def _build_tables_kernel():
    @functools.partial(
        pl.kernel,
        out_type=(
            jax.ShapeDtypeStruct((SLOTS,), jnp.int32),
            jax.ShapeDtypeStruct((SLOTS,), jnp.float32),
        ),
        mesh=_mesh(),
        compiler_params=pltpu.CompilerParams(needs_layout_passes=False),
        scratch_types=[
            pltpu.VMEM((T,), jnp.int32),
            pltpu.VMEM((T,), jnp.float32),
            pltpu.VMEM((SLOTS,), jnp.int32),
            pltpu.VMEM((SLOTS,), jnp.float32),
        ],
    )
    def _build_tables(slot_hbm, prob_hbm, idx_hbm, pbuf_hbm,
                      slot_v, prob_v, idx_v, pbuf_v):
        wid = lax.axis_index("s") * NC + lax.axis_index("c")

        @pl.when(wid == 0)
        def _():
            pltpu.sync_copy(slot_hbm, slot_v)
            pltpu.sync_copy(prob_hbm, prob_v)

            zi = jnp.zeros((16,), jnp.int32)
            zf = jnp.zeros((16,), jnp.float32)

            def init(i, carry):
                idx_v[pl.ds(i * 16, 16)] = zi
                pbuf_v[pl.ds(i * 16, 16)] = zf
                return carry

            lax.fori_loop(0, SLOTS // 16, init, 0)

            def body(i, carry):
                s = slot_v[pl.ds(i * 16, 16)]
                p = prob_v[pl.ds(i * 16, 16)]
                t = lax.iota(jnp.int32, 16) + i * 16
                msk = s < SENT
                plsc.store_scatter(idx_v, [s], t, mask=msk)
                plsc.store_scatter(pbuf_v, [s], p, mask=msk)
                return carry

            lax.fori_loop(0, T // 16, body, 0)
            pltpu.sync_copy(idx_v, idx_hbm)
            pltpu.sync_copy(pbuf_v, pbuf_hbm)

    return _build_tables


D_PER_W = SLOTS // NW
D_CHUNK = 88


@functools.cache
def _dispatch_kernel():
    @functools.partial(
        pl.kernel,
        out_type=jax.ShapeDtypeStruct((SLOTS, H), jnp.float32),
        mesh=_mesh(),
        compiler_params=pltpu.CompilerParams(needs_layout_passes=False),
        scratch_types=[
            pltpu.VMEM((D_CHUNK,), jnp.int32),
            pltpu.VMEM((D_CHUNK, H), jnp.float32),
            pltpu.SemaphoreType.DMA,
        ],
    )
    def _dispatch(idx_hbm, hid_hbm, xbuf_hbm, idx_v, rows_v, sem):
        wid = lax.axis_index("s") * NC + lax.axis_index("c")
        for j in range(D_PER_W // D_CHUNK):
            base = wid * D_PER_W + j * D_CHUNK
            pltpu.sync_copy(idx_hbm.at[pl.ds(base, D_CHUNK)], idx_v)
            pltpu.async_copy(hid_hbm.at[idx_v], rows_v, sem).wait()
            pltpu.sync_copy(rows_v, xbuf_hbm.at[pl.ds(base, D_CHUNK)])

    return _dispatch


def _expert_body(x_ref, w1_ref, w2_ref, p_ref, y_ref):
    x = x_ref[...]
    h = jnp.dot(x, w1_ref[...], preferred_element_type=jnp.float32)
    a = h * lax.logistic(h)
    y = jnp.dot(a, w2_ref[...], preferred_element_type=jnp.float32)
    y_ref[...] = y * p_ref[...]


def _expert_gemms(xbuf, w1, w2, pbuf_col):
    return pl.pallas_call(
        _expert_body,
        grid=(GRID_E,),
        in_specs=[
            pl.BlockSpec((CAP, H), lambda e: (e, 0)),
            pl.BlockSpec((H, F), lambda e: (0, jnp.minimum(e, E - 1))),
            pl.BlockSpec((F, H), lambda e: (jnp.minimum(e, E - 1), 0)),
            pl.BlockSpec((CAP, 1), lambda e: (e, 0)),
        ],
        out_specs=pl.BlockSpec((CAP, H), lambda e: (e, 0)),
        out_shape=jax.ShapeDtypeStruct((SLOTS, H), jnp.float32),
    )(xbuf, w1, w2, pbuf_col)


C_PER_W = T // NW
C_CHUNK = 64


@functools.cache
def _combine_kernel():
    @functools.partial(
        pl.kernel,
        out_type=jax.ShapeDtypeStruct((T, H), jnp.float32),
        mesh=_mesh(),
        compiler_params=pltpu.CompilerParams(needs_layout_passes=False),
        scratch_types=[
            pltpu.VMEM((C_CHUNK,), jnp.int32),
            pltpu.VMEM((C_CHUNK, H), jnp.float32),
            pltpu.SemaphoreType.DMA,
        ],
    )
    def _combine(slot_hbm, ybuf_hbm, out_hbm, slot_v, rows_v, sem):
        wid = lax.axis_index("s") * NC + lax.axis_index("c")
        for j in range(C_PER_W // C_CHUNK):
            base = wid * C_PER_W + j * C_CHUNK
            pltpu.sync_copy(slot_hbm.at[pl.ds(base, C_CHUNK)], slot_v)
            pltpu.async_copy(ybuf_hbm.at[slot_v], rows_v, sem).wait()
            pltpu.sync_copy(rows_v, out_hbm.at[pl.ds(base, C_CHUNK)])

    return _combine


def kernel(hidden_states, router_weight, weight1, weight2):
    slot3, prob3 = _router(hidden_states, router_weight)
    slot = slot3.reshape(T)
    prob = prob3.reshape(T)
    idx, pbuf = _build_tables_kernel()(slot, prob)
    xbuf = _dispatch_kernel()(idx, hidden_states)
    ybuf = _expert_gemms(xbuf, weight1, weight2, pbuf.reshape(SLOTS, 1))
    return _combine_kernel()(slot, ybuf)


# speedup vs baseline: 1.0240x; 1.0240x over previous
import functools

import jax
import jax.numpy as jnp
from jax import lax
from jax.experimental import pallas as pl
from jax.experimental.pallas import tpu as pltpu
from jax.experimental.pallas import tpu_sc as plsc

E = 64
H = 1024
F = 512
T = 4096
CAP = 128
BT = 512
NB = T // BT
SENT = E * CAP
GRID_E = 66
SLOTS = GRID_E * CAP

NC, NS = 2, 16
NW = NC * NS


def _router_body(x_ref, rw_ref, slot_ref, prob_ref, counts_ref):
    b = pl.program_id(0)

    @pl.when(b == 0)
    def _():
        counts_ref[...] = jnp.zeros_like(counts_ref)

    x = x_ref[...]
    rw = rw_ref[...]
    logits = jnp.dot(x, rw, preferred_element_type=jnp.float32)
    m = jnp.max(logits, axis=1, keepdims=True)
    denom = jnp.sum(jnp.exp(logits - m), axis=1)
    prob = 1.0 / denom

    ids = lax.broadcasted_iota(jnp.int32, (BT, E), 1)
    is_max = logits == m
    expert = jnp.min(jnp.where(is_max, ids, E), axis=1)
    onehot = (ids == expert[:, None]).astype(jnp.float32)

    ti = lax.broadcasted_iota(jnp.int32, (BT, BT), 0)
    tj = lax.broadcasted_iota(jnp.int32, (BT, BT), 1)
    tri = (tj < ti).astype(jnp.float32)
    rank_mat = jnp.dot(tri, onehot, preferred_element_type=jnp.float32)
    rank = jnp.sum(rank_mat * onehot, axis=1)
    base = jnp.sum(counts_ref[...] * onehot, axis=1)
    pos = base + rank
    valid = pos < CAP
    slot = jnp.where(valid, expert * CAP + pos.astype(jnp.int32), SENT)
    counts_ref[...] = counts_ref[...] + jnp.sum(onehot, axis=0, keepdims=True)

    slot_ref[...] = slot.reshape(1, 1, BT)
    prob_ref[...] = prob.reshape(1, 1, BT)


def _router(hidden, rw):
    return pl.pallas_call(
        _router_body,
        grid=(NB,),
        in_specs=[
            pl.BlockSpec((BT, H), lambda b: (b, 0)),
            pl.BlockSpec((H, E), lambda b: (0, 0)),
        ],
        out_specs=[
            pl.BlockSpec((1, 1, BT), lambda b: (b, 0, 0)),
            pl.BlockSpec((1, 1, BT), lambda b: (b, 0, 0)),
        ],
        out_shape=[
            jax.ShapeDtypeStruct((NB, 1, BT), jnp.int32),
            jax.ShapeDtypeStruct((NB, 1, BT), jnp.float32),
        ],
        scratch_shapes=[pltpu.VMEM((1, E), jnp.float32)],
    )(hidden, rw)


def _mesh():
    return plsc.VectorSubcoreMesh(
        core_axis_name="c", subcore_axis_name="s", num_cores=NC, num_subcores=NS
    )


@functools.cache
def _build_tables_kernel():
    @functools.partial(
        pl.kernel,
        out_type=(
            jax.ShapeDtypeStruct((SLOTS,), jnp.int32),
            jax.ShapeDtypeStruct((SLOTS,), jnp.float32),
        ),
        mesh=_mesh(),
        compiler_params=pltpu.CompilerParams(needs_layout_passes=False),
        scratch_types=[
            pltpu.VMEM((T,), jnp.int32),
            pltpu.VMEM((T,), jnp.float32),
            pltpu.VMEM((SLOTS,), jnp.int32),
            pltpu.VMEM((SLOTS,), jnp.float32),
        ],
    )
    def _build_tables(slot_hbm, prob_hbm, idx_hbm, pbuf_hbm,
                      slot_v, prob_v, idx_v, pbuf_v):
        wid = lax.axis_index("s") * NC + lax.axis_index("c")

        @pl.when(wid == 0)
        def _():
            pltpu.sync_copy(slot_hbm, slot_v)
            pltpu.sync_copy(prob_hbm, prob_v)

            zi = jnp.zeros((16,), jnp.int32)
            zf = jnp.zeros((16,), jnp.float32)

            def init(i, carry):
                idx_v[pl.ds(i * 16, 16)] = zi
                pbuf_v[pl.ds(i * 16, 16)] = zf
                return carry

            lax.fori_loop(0, SLOTS // 16, init, 0)

            def body(i, carry):
                s = slot_v[pl.ds(i * 16, 16)]
                p = prob_v[pl.ds(i * 16, 16)]
                t = lax.iota(jnp.int32, 16) + i * 16
                msk = s < SENT
                plsc.store_scatter(idx_v, [s], t, mask=msk)
                plsc.store_scatter(pbuf_v, [s], p, mask=msk)
                return carry

            lax.fori_loop(0, T // 16, body, 0)
            pltpu.sync_copy(idx_v, idx_hbm)
            pltpu.sync_copy(pbuf_v, pbuf_hbm)

    return _build_tables


D_CHUNK = 64
D_NCHUNK = SLOTS // D_CHUNK


@functools.cache
def _dispatch_kernel():
    @functools.partial(
        pl.kernel,
        out_type=jax.ShapeDtypeStruct((SLOTS, H), jnp.float32),
        mesh=_mesh(),
        compiler_params=pltpu.CompilerParams(needs_layout_passes=False),
        scratch_types=[
            pltpu.VMEM((D_CHUNK,), jnp.int32),
            pltpu.VMEM((D_CHUNK, H), jnp.float32),
            pltpu.SemaphoreType.DMA,
        ],
    )
    def _dispatch(idx_hbm, hid_hbm, xbuf_hbm, idx_v, rows_v, sem):
        wid = lax.axis_index("s") * NC + lax.axis_index("c")

        def chunk(cid):
            base = cid * D_CHUNK
            pltpu.sync_copy(idx_hbm.at[pl.ds(base, D_CHUNK)], idx_v)
            pltpu.async_copy(hid_hbm.at[idx_v], rows_v, sem).wait()
            pltpu.sync_copy(rows_v, xbuf_hbm.at[pl.ds(base, D_CHUNK)])

        for j in range(4):
            chunk(wid * 4 + j)

        @pl.when(wid < D_NCHUNK - 4 * NW)
        def _():
            chunk(4 * NW + wid)

    return _dispatch


def _expert_body(x_ref, w1_ref, w2_ref, p_ref, y_ref):
    x = x_ref[...]
    h = jnp.dot(x, w1_ref[...], preferred_element_type=jnp.float32)
    a = h * lax.logistic(h)
    y = jnp.dot(a, w2_ref[...], preferred_element_type=jnp.float32)
    y_ref[...] = y * p_ref[...]


def _expert_gemms(xbuf, w1, w2, pbuf_col):
    return pl.pallas_call(
        _expert_body,
        grid=(GRID_E,),
        in_specs=[
            pl.BlockSpec((CAP, H), lambda e: (e, 0)),
            pl.BlockSpec((H, F), lambda e: (0, jnp.minimum(e, E - 1))),
            pl.BlockSpec((F, H), lambda e: (jnp.minimum(e, E - 1), 0)),
            pl.BlockSpec((CAP, 1), lambda e: (e, 0)),
        ],
        out_specs=pl.BlockSpec((CAP, H), lambda e: (e, 0)),
        out_shape=jax.ShapeDtypeStruct((SLOTS, H), jnp.float32),
    )(xbuf, w1, w2, pbuf_col)


C_PER_W = T // NW
C_CHUNK = 64


@functools.cache
def _combine_kernel():
    @functools.partial(
        pl.kernel,
        out_type=jax.ShapeDtypeStruct((T, H), jnp.float32),
        mesh=_mesh(),
        compiler_params=pltpu.CompilerParams(needs_layout_passes=False),
        scratch_types=[
            pltpu.VMEM((C_CHUNK,), jnp.int32),
            pltpu.VMEM((C_CHUNK, H), jnp.float32),
            pltpu.SemaphoreType.DMA,
        ],
    )
    def _combine(slot_hbm, ybuf_hbm, out_hbm, slot_v, rows_v, sem):
        wid = lax.axis_index("s") * NC + lax.axis_index("c")
        for j in range(C_PER_W // C_CHUNK):
            base = wid * C_PER_W + j * C_CHUNK
            pltpu.sync_copy(slot_hbm.at[pl.ds(base, C_CHUNK)], slot_v)
            pltpu.async_copy(ybuf_hbm.at[slot_v], rows_v, sem).wait()
            pltpu.sync_copy(rows_v, out_hbm.at[pl.ds(base, C_CHUNK)])

    return _combine


def kernel(hidden_states, router_weight, weight1, weight2):
    slot3, prob3 = _router(hidden_states, router_weight)
    slot = slot3.reshape(T)
    prob = prob3.reshape(T)
    idx, pbuf = _build_tables_kernel()(slot, prob)
    xbuf = _dispatch_kernel()(idx, hidden_states)
    ybuf = _expert_gemms(xbuf, weight1, weight2, pbuf.reshape(SLOTS, 1))
    return _combine_kernel()(slot, ybuf)


# speedup vs baseline: 1.9168x; 1.8720x over previous
import functools

import jax
import jax.numpy as jnp
from jax import lax
from jax.experimental import pallas as pl
from jax.experimental.pallas import tpu as pltpu
from jax.experimental.pallas import tpu_sc as plsc

E = 64
H = 1024
F = 512
T = 4096
CAP = 128
BT = 512
NB = T // BT
SENT = E * CAP
GRID_E = 66
SLOTS = GRID_E * CAP

NC, NS = 2, 16
NW = NC * NS


def _router_body(x_ref, rw_ref, slot_ref, prob_ref, counts_ref):
    b = pl.program_id(0)

    @pl.when(b == 0)
    def _():
        counts_ref[...] = jnp.zeros_like(counts_ref)

    x = x_ref[...]
    rw = rw_ref[...]
    logits = jnp.dot(x, rw, preferred_element_type=jnp.float32)
    m = jnp.max(logits, axis=1, keepdims=True)
    denom = jnp.sum(jnp.exp(logits - m), axis=1)
    prob = 1.0 / denom

    ids = lax.broadcasted_iota(jnp.int32, (BT, E), 1)
    is_max = logits == m
    expert = jnp.min(jnp.where(is_max, ids, E), axis=1)
    onehot = (ids == expert[:, None]).astype(jnp.float32)

    ti = lax.broadcasted_iota(jnp.int32, (BT, BT), 0)
    tj = lax.broadcasted_iota(jnp.int32, (BT, BT), 1)
    tri = (tj < ti).astype(jnp.float32)
    rank_mat = jnp.dot(tri, onehot, preferred_element_type=jnp.float32)
    rank = jnp.sum(rank_mat * onehot, axis=1)
    base = jnp.sum(counts_ref[...] * onehot, axis=1)
    pos = base + rank
    valid = pos < CAP
    slot = jnp.where(valid, expert * CAP + pos.astype(jnp.int32), SENT)
    counts_ref[...] = counts_ref[...] + jnp.sum(onehot, axis=0, keepdims=True)

    slot_ref[...] = slot.reshape(1, 1, BT)
    prob_ref[...] = prob.reshape(1, 1, BT)


def _router(hidden, rw):
    return pl.pallas_call(
        _router_body,
        grid=(NB,),
        in_specs=[
            pl.BlockSpec((BT, H), lambda b: (b, 0)),
            pl.BlockSpec((H, E), lambda b: (0, 0)),
        ],
        out_specs=[
            pl.BlockSpec((1, 1, BT), lambda b: (b, 0, 0)),
            pl.BlockSpec((1, 1, BT), lambda b: (b, 0, 0)),
        ],
        out_shape=[
            jax.ShapeDtypeStruct((NB, 1, BT), jnp.int32),
            jax.ShapeDtypeStruct((NB, 1, BT), jnp.float32),
        ],
        scratch_shapes=[pltpu.VMEM((1, E), jnp.float32)],
    )(hidden, rw)


def _mesh():
    return plsc.VectorSubcoreMesh(
        core_axis_name="c", subcore_axis_name="s", num_cores=NC, num_subcores=NS
    )


@functools.cache
def _build_tables_kernel():
    @functools.partial(
        pl.kernel,
        out_type=(
            jax.ShapeDtypeStruct((SLOTS,), jnp.int32),
            jax.ShapeDtypeStruct((SLOTS,), jnp.float32),
        ),
        mesh=_mesh(),
        compiler_params=pltpu.CompilerParams(needs_layout_passes=False),
        scratch_types=[
            pltpu.VMEM((T,), jnp.int32),
            pltpu.VMEM((T,), jnp.float32),
            pltpu.VMEM((SLOTS,), jnp.int32),
            pltpu.VMEM((SLOTS,), jnp.float32),
        ],
    )
    def _build_tables(slot_hbm, prob_hbm, idx_hbm, pbuf_hbm,
                      slot_v, prob_v, idx_v, pbuf_v):
        wid = lax.axis_index("s") * NC + lax.axis_index("c")

        @pl.when(wid == 0)
        def _():
            pltpu.sync_copy(slot_hbm, slot_v)
            pltpu.sync_copy(prob_hbm, prob_v)

            zf = jnp.zeros((16,), jnp.float32)

            def init(i, carry):
                idx_v[pl.ds(i * 16, 16)] = (lax.iota(jnp.int32, 16) + i * 16) & (T - 1)
                pbuf_v[pl.ds(i * 16, 16)] = zf
                return carry

            lax.fori_loop(0, SLOTS // 16, init, 0)

            def body(i, carry):
                s = slot_v[pl.ds(i * 16, 16)]
                p = prob_v[pl.ds(i * 16, 16)]
                t = lax.iota(jnp.int32, 16) + i * 16
                msk = s < SENT
                plsc.store_scatter(idx_v, [s], t, mask=msk)
                plsc.store_scatter(pbuf_v, [s], p, mask=msk)
                return carry

            lax.fori_loop(0, T // 16, body, 0)
            pltpu.sync_copy(idx_v, idx_hbm)
            pltpu.sync_copy(pbuf_v, pbuf_hbm)

    return _build_tables


D_CHUNK = 64
D_NCHUNK = SLOTS // D_CHUNK


@functools.cache
def _dispatch_kernel():
    @functools.partial(
        pl.kernel,
        out_type=jax.ShapeDtypeStruct((SLOTS, H), jnp.float32),
        mesh=_mesh(),
        compiler_params=pltpu.CompilerParams(needs_layout_passes=False),
        scratch_types=[
            pltpu.VMEM((D_CHUNK,), jnp.int32),
            pltpu.VMEM((D_CHUNK, H), jnp.float32),
            pltpu.SemaphoreType.DMA,
        ],
    )
    def _dispatch(idx_hbm, hid_hbm, xbuf_hbm, idx_v, rows_v, sem):
        wid = lax.axis_index("s") * NC + lax.axis_index("c")

        def chunk(cid):
            base = cid * D_CHUNK
            pltpu.sync_copy(idx_hbm.at[pl.ds(base, D_CHUNK)], idx_v)
            pltpu.async_copy(hid_hbm.at[idx_v], rows_v, sem).wait()
            pltpu.sync_copy(rows_v, xbuf_hbm.at[pl.ds(base, D_CHUNK)])

        for j in range(4):
            chunk(wid * 4 + j)

        @pl.when(wid < D_NCHUNK - 4 * NW)
        def _():
            chunk(4 * NW + wid)

    return _dispatch


def _expert_body(x_ref, w1_ref, w2_ref, p_ref, y_ref):
    x = x_ref[...]
    h = jnp.dot(x, w1_ref[...], preferred_element_type=jnp.float32)
    a = h * lax.logistic(h)
    y = jnp.dot(a, w2_ref[...], preferred_element_type=jnp.float32)
    y_ref[...] = y * p_ref[...]


def _expert_gemms(xbuf, w1, w2, pbuf_col):
    return pl.pallas_call(
        _expert_body,
        grid=(GRID_E,),
        in_specs=[
            pl.BlockSpec((CAP, H), lambda e: (e, 0)),
            pl.BlockSpec((H, F), lambda e: (0, jnp.minimum(e, E - 1))),
            pl.BlockSpec((F, H), lambda e: (jnp.minimum(e, E - 1), 0)),
            pl.BlockSpec((CAP, 1), lambda e: (e, 0)),
        ],
        out_specs=pl.BlockSpec((CAP, H), lambda e: (e, 0)),
        out_shape=jax.ShapeDtypeStruct((SLOTS, H), jnp.float32),
    )(xbuf, w1, w2, pbuf_col)


C_PER_W = T // NW
C_CHUNK = 64


@functools.cache
def _combine_kernel():
    @functools.partial(
        pl.kernel,
        out_type=jax.ShapeDtypeStruct((T, H), jnp.float32),
        mesh=_mesh(),
        compiler_params=pltpu.CompilerParams(needs_layout_passes=False),
        scratch_types=[
            pltpu.VMEM((C_CHUNK,), jnp.int32),
            pltpu.VMEM((C_CHUNK, H), jnp.float32),
            pltpu.SemaphoreType.DMA,
        ],
    )
    def _combine(slot_hbm, ybuf_hbm, out_hbm, slot_v, rows_v, sem):
        wid = lax.axis_index("s") * NC + lax.axis_index("c")
        for j in range(C_PER_W // C_CHUNK):
            base = wid * C_PER_W + j * C_CHUNK
            pltpu.sync_copy(slot_hbm.at[pl.ds(base, C_CHUNK)], slot_v)
            pltpu.async_copy(ybuf_hbm.at[slot_v], rows_v, sem).wait()
            pltpu.sync_copy(rows_v, out_hbm.at[pl.ds(base, C_CHUNK)])

    return _combine


def kernel(hidden_states, router_weight, weight1, weight2):
    slot3, prob3 = _router(hidden_states, router_weight)
    slot = slot3.reshape(T)
    prob = prob3.reshape(T)
    idx, pbuf = _build_tables_kernel()(slot, prob)
    xbuf = _dispatch_kernel()(idx, hidden_states)
    ybuf = _expert_gemms(xbuf, weight1, weight2, pbuf.reshape(SLOTS, 1))
    return _combine_kernel()(slot, ybuf)


# speedup vs baseline: 2.0220x; 1.0549x over previous
import functools

import jax
import jax.numpy as jnp
from jax import lax
from jax.experimental import pallas as pl
from jax.experimental.pallas import tpu as pltpu
from jax.experimental.pallas import tpu_sc as plsc

E = 64
H = 1024
F = 512
T = 4096
CAP = 128
BT = 512
NB = T // BT
SENT = E * CAP
GRID_E = 66
SLOTS = GRID_E * CAP

NC, NS = 2, 16
NW = NC * NS


def _router_body(x_ref, rw_ref, slot_ref, prob_ref, counts_ref):
    b = pl.program_id(0)

    @pl.when(b == 0)
    def _():
        counts_ref[...] = jnp.zeros_like(counts_ref)

    x = x_ref[...]
    rw = rw_ref[...]
    logits = jnp.dot(x, rw, preferred_element_type=jnp.float32)
    m = jnp.max(logits, axis=1, keepdims=True)
    denom = jnp.sum(jnp.exp(logits - m), axis=1)
    prob = 1.0 / denom

    ids = lax.broadcasted_iota(jnp.int32, (BT, E), 1)
    is_max = logits == m
    expert = jnp.min(jnp.where(is_max, ids, E), axis=1)
    onehot = (ids == expert[:, None]).astype(jnp.float32)

    ti = lax.broadcasted_iota(jnp.int32, (BT, BT), 0)
    tj = lax.broadcasted_iota(jnp.int32, (BT, BT), 1)
    tri = (tj < ti).astype(jnp.float32)
    rank_mat = jnp.dot(tri, onehot, preferred_element_type=jnp.float32)
    rank = jnp.sum(rank_mat * onehot, axis=1)
    base = jnp.sum(counts_ref[...] * onehot, axis=1)
    pos = base + rank
    valid = pos < CAP
    slot = jnp.where(valid, expert * CAP + pos.astype(jnp.int32), SENT)
    counts_ref[...] = counts_ref[...] + jnp.sum(onehot, axis=0, keepdims=True)

    slot_ref[...] = slot
    prob_ref[...] = prob


def _router(hidden, rw):
    return pl.pallas_call(
        _router_body,
        grid=(NB,),
        in_specs=[
            pl.BlockSpec((BT, H), lambda b: (b, 0)),
            pl.BlockSpec((H, E), lambda b: (0, 0)),
        ],
        out_specs=[
            pl.BlockSpec((BT,), lambda b: (b,)),
            pl.BlockSpec((BT,), lambda b: (b,)),
        ],
        out_shape=[
            jax.ShapeDtypeStruct((T,), jnp.int32),
            jax.ShapeDtypeStruct((T,), jnp.float32),
        ],
        scratch_shapes=[pltpu.VMEM((1, E), jnp.float32)],
    )(hidden, rw)


def _mesh():
    return plsc.VectorSubcoreMesh(
        core_axis_name="c", subcore_axis_name="s", num_cores=NC, num_subcores=NS
    )


@functools.cache
def _build_tables_kernel():
    @functools.partial(
        pl.kernel,
        out_type=(
            jax.ShapeDtypeStruct((SLOTS,), jnp.int32),
            jax.ShapeDtypeStruct((SLOTS,), jnp.float32),
        ),
        mesh=_mesh(),
        compiler_params=pltpu.CompilerParams(needs_layout_passes=False),
        scratch_types=[
            pltpu.VMEM((T,), jnp.int32),
            pltpu.VMEM((T,), jnp.float32),
            pltpu.VMEM((SLOTS,), jnp.int32),
            pltpu.VMEM((SLOTS,), jnp.float32),
        ],
    )
    def _build_tables(slot_hbm, prob_hbm, idx_hbm, pbuf_hbm,
                      slot_v, prob_v, idx_v, pbuf_v):
        wid = lax.axis_index("s") * NC + lax.axis_index("c")

        @pl.when(wid == 0)
        def _():
            pltpu.sync_copy(slot_hbm, slot_v)
            pltpu.sync_copy(prob_hbm, prob_v)

            zf = jnp.zeros((16,), jnp.float32)

            def init(i, carry):
                idx_v[pl.ds(i * 16, 16)] = (lax.iota(jnp.int32, 16) + i * 16) & (T - 1)
                pbuf_v[pl.ds(i * 16, 16)] = zf
                return carry

            lax.fori_loop(0, SLOTS // 16, init, 0)

            def body(i, carry):
                s = slot_v[pl.ds(i * 16, 16)]
                p = prob_v[pl.ds(i * 16, 16)]
                t = lax.iota(jnp.int32, 16) + i * 16
                msk = s < SENT
                plsc.store_scatter(idx_v, [s], t, mask=msk)
                plsc.store_scatter(pbuf_v, [s], p, mask=msk)
                return carry

            lax.fori_loop(0, T // 16, body, 0)
            pltpu.sync_copy(idx_v, idx_hbm)
            pltpu.sync_copy(pbuf_v, pbuf_hbm)

    return _build_tables


D_CHUNK = 64
D_NCHUNK = SLOTS // D_CHUNK


@functools.cache
def _dispatch_kernel():
    @functools.partial(
        pl.kernel,
        out_type=jax.ShapeDtypeStruct((SLOTS, H), jnp.float32),
        mesh=_mesh(),
        compiler_params=pltpu.CompilerParams(needs_layout_passes=False),
        scratch_types=[
            pltpu.VMEM((D_CHUNK,), jnp.int32),
            pltpu.VMEM((D_CHUNK, H), jnp.float32),
            pltpu.SemaphoreType.DMA,
        ],
    )
    def _dispatch(idx_hbm, hid_hbm, xbuf_hbm, idx_v, rows_v, sem):
        wid = lax.axis_index("s") * NC + lax.axis_index("c")

        def chunk(cid):
            base = cid * D_CHUNK
            pltpu.sync_copy(idx_hbm.at[pl.ds(base, D_CHUNK)], idx_v)
            pltpu.async_copy(hid_hbm.at[idx_v], rows_v, sem).wait()
            pltpu.sync_copy(rows_v, xbuf_hbm.at[pl.ds(base, D_CHUNK)])

        for j in range(4):
            chunk(wid * 4 + j)

        @pl.when(wid < D_NCHUNK - 4 * NW)
        def _():
            chunk(4 * NW + wid)

    return _dispatch


def _expert_body(x_ref, w1_ref, w2_ref, p_ref, y_ref):
    for k in range(2):
        r = pl.ds(k * CAP, CAP)
        c = pl.ds(k * F, F)
        x = x_ref[r, :]
        h = jnp.dot(x, w1_ref[:, c], preferred_element_type=jnp.float32)
        a = h * lax.logistic(h)
        y = jnp.dot(a, w2_ref[c, :], preferred_element_type=jnp.float32)
        y_ref[r, :] = y * p_ref[r, :]


def _expert_gemms(xbuf, w1, w2, pbuf_col):
    return pl.pallas_call(
        _expert_body,
        grid=(GRID_E // 2,),
        in_specs=[
            pl.BlockSpec((2 * CAP, H), lambda e: (e, 0)),
            pl.BlockSpec((H, 2 * F), lambda e: (0, jnp.minimum(e, E // 2 - 1))),
            pl.BlockSpec((2 * F, H), lambda e: (jnp.minimum(e, E // 2 - 1), 0)),
            pl.BlockSpec((2 * CAP, 1), lambda e: (e, 0)),
        ],
        out_specs=pl.BlockSpec((2 * CAP, H), lambda e: (e, 0)),
        out_shape=jax.ShapeDtypeStruct((SLOTS, H), jnp.float32),
    )(xbuf, w1, w2, pbuf_col)


C_PER_W = T // NW
C_CHUNK = 64


@functools.cache
def _combine_kernel():
    @functools.partial(
        pl.kernel,
        out_type=jax.ShapeDtypeStruct((T, H), jnp.float32),
        mesh=_mesh(),
        compiler_params=pltpu.CompilerParams(needs_layout_passes=False),
        scratch_types=[
            pltpu.VMEM((C_CHUNK,), jnp.int32),
            pltpu.VMEM((C_CHUNK, H), jnp.float32),
            pltpu.SemaphoreType.DMA,
        ],
    )
    def _combine(slot_hbm, ybuf_hbm, out_hbm, slot_v, rows_v, sem):
        wid = lax.axis_index("s") * NC + lax.axis_index("c")
        for j in range(C_PER_W // C_CHUNK):
            base = wid * C_PER_W + j * C_CHUNK
            pltpu.sync_copy(slot_hbm.at[pl.ds(base, C_CHUNK)], slot_v)
            pltpu.async_copy(ybuf_hbm.at[slot_v], rows_v, sem).wait()
            pltpu.sync_copy(rows_v, out_hbm.at[pl.ds(base, C_CHUNK)])

    return _combine


def kernel(hidden_states, router_weight, weight1, weight2):
    slot, prob = _router(hidden_states, router_weight)
    idx, pbuf = _build_tables_kernel()(slot, prob)
    xbuf = _dispatch_kernel()(idx, hidden_states)
    ybuf = _expert_gemms(xbuf, weight1, weight2, pbuf.reshape(SLOTS, 1))
    return _combine_kernel()(slot, ybuf)


# speedup vs baseline: 2.1521x; 1.0644x over previous
import functools

import jax
import jax.numpy as jnp
from jax import lax
from jax.experimental import pallas as pl
from jax.experimental.pallas import tpu as pltpu
from jax.experimental.pallas import tpu_sc as plsc

E = 64
H = 1024
F = 512
T = 4096
CAP = 128
BT = 512
NB = T // BT
SENT = E * CAP
GRID_E = 66
SLOTS = GRID_E * CAP

NC, NS = 2, 16
NW = NC * NS


def _router_body(x_ref, rw_ref, slot_ref, prob_ref, counts_ref):
    b = pl.program_id(0)

    @pl.when(b == 0)
    def _():
        counts_ref[...] = jnp.zeros_like(counts_ref)

    x = x_ref[...]
    rw = rw_ref[...]
    logits = lax.dot_general(
        rw, x, dimension_numbers=(((0,), (1,)), ((), ())),
        preferred_element_type=jnp.float32)
    m = jnp.max(logits, axis=0, keepdims=True)
    denom = jnp.sum(jnp.exp(logits - m), axis=0)
    prob = 1.0 / denom

    ids = lax.broadcasted_iota(jnp.int32, (E, BT), 0)
    is_max = logits == m
    expert = jnp.min(jnp.where(is_max, ids, E), axis=0)
    onehot = (ids == expert[None, :]).astype(jnp.float32)

    tj = lax.broadcasted_iota(jnp.int32, (BT, BT), 0)
    ti = lax.broadcasted_iota(jnp.int32, (BT, BT), 1)
    tri = (tj < ti).astype(jnp.float32)
    rank_mat = jnp.dot(onehot, tri, preferred_element_type=jnp.float32)
    rank = jnp.sum(rank_mat * onehot, axis=0)
    base = jnp.sum(counts_ref[...] * onehot, axis=0)
    pos = base + rank
    valid = pos < CAP
    slot = jnp.where(valid, expert * CAP + pos.astype(jnp.int32), SENT)
    counts_ref[...] = counts_ref[...] + jnp.sum(onehot, axis=1, keepdims=True)

    slot_ref[...] = slot
    prob_ref[...] = prob


def _router(hidden, rw):
    return pl.pallas_call(
        _router_body,
        grid=(NB,),
        in_specs=[
            pl.BlockSpec((BT, H), lambda b: (b, 0)),
            pl.BlockSpec((H, E), lambda b: (0, 0)),
        ],
        out_specs=[
            pl.BlockSpec((BT,), lambda b: (b,)),
            pl.BlockSpec((BT,), lambda b: (b,)),
        ],
        out_shape=[
            jax.ShapeDtypeStruct((T,), jnp.int32),
            jax.ShapeDtypeStruct((T,), jnp.float32),
        ],
        scratch_shapes=[pltpu.VMEM((E, 1), jnp.float32)],
    )(hidden, rw)


def _mesh():
    return plsc.VectorSubcoreMesh(
        core_axis_name="c", subcore_axis_name="s", num_cores=NC, num_subcores=NS
    )


@functools.cache
def _build_tables_kernel():
    @functools.partial(
        pl.kernel,
        out_type=(
            jax.ShapeDtypeStruct((SLOTS,), jnp.int32),
            jax.ShapeDtypeStruct((SLOTS,), jnp.float32),
        ),
        mesh=_mesh(),
        compiler_params=pltpu.CompilerParams(needs_layout_passes=False),
        scratch_types=[
            pltpu.VMEM((T,), jnp.int32),
            pltpu.VMEM((T,), jnp.float32),
            pltpu.VMEM((SLOTS,), jnp.int32),
            pltpu.VMEM((SLOTS,), jnp.float32),
        ],
    )
    def _build_tables(slot_hbm, prob_hbm, idx_hbm, pbuf_hbm,
                      slot_v, prob_v, idx_v, pbuf_v):
        wid = lax.axis_index("s") * NC + lax.axis_index("c")

        @pl.when(wid == 0)
        def _():
            pltpu.sync_copy(slot_hbm, slot_v)
            pltpu.sync_copy(prob_hbm, prob_v)

            zf = jnp.zeros((16,), jnp.float32)

            def init(i, carry):
                idx_v[pl.ds(i * 16, 16)] = (lax.iota(jnp.int32, 16) + i * 16) & (T - 1)
                pbuf_v[pl.ds(i * 16, 16)] = zf
                return carry

            lax.fori_loop(0, SLOTS // 16, init, 0)

            def body(i, carry):
                s = slot_v[pl.ds(i * 16, 16)]
                p = prob_v[pl.ds(i * 16, 16)]
                t = lax.iota(jnp.int32, 16) + i * 16
                msk = s < SENT
                plsc.store_scatter(idx_v, [s], t, mask=msk)
                plsc.store_scatter(pbuf_v, [s], p, mask=msk)
                return carry

            lax.fori_loop(0, T // 16, body, 0)
            pltpu.sync_copy(idx_v, idx_hbm)
            pltpu.sync_copy(pbuf_v, pbuf_hbm)

    return _build_tables


D_CHUNK = 64
D_NCHUNK = SLOTS // D_CHUNK


@functools.cache
def _dispatch_kernel():
    @functools.partial(
        pl.kernel,
        out_type=jax.ShapeDtypeStruct((SLOTS, H), jnp.float32),
        mesh=_mesh(),
        compiler_params=pltpu.CompilerParams(needs_layout_passes=False),
        scratch_types=[
            pltpu.VMEM((D_CHUNK,), jnp.int32),
            pltpu.VMEM((D_CHUNK, H), jnp.float32),
            pltpu.SemaphoreType.DMA,
        ],
    )
    def _dispatch(idx_hbm, hid_hbm, xbuf_hbm, idx_v, rows_v, sem):
        wid = lax.axis_index("s") * NC + lax.axis_index("c")

        def chunk(cid):
            base = cid * D_CHUNK
            pltpu.sync_copy(idx_hbm.at[pl.ds(base, D_CHUNK)], idx_v)
            pltpu.async_copy(hid_hbm.at[idx_v], rows_v, sem).wait()
            pltpu.sync_copy(rows_v, xbuf_hbm.at[pl.ds(base, D_CHUNK)])

        for j in range(4):
            chunk(wid * 4 + j)

        @pl.when(wid < D_NCHUNK - 4 * NW)
        def _():
            chunk(4 * NW + wid)

    return _dispatch


def _expert_body(x_ref, w1_ref, w2_ref, p_ref, y_ref):
    for k in range(2):
        r = pl.ds(k * CAP, CAP)
        c = pl.ds(k * F, F)
        x = x_ref[r, :]
        h = jnp.dot(x, w1_ref[:, c], preferred_element_type=jnp.float32)
        a = h * lax.logistic(h)
        y = jnp.dot(a, w2_ref[c, :], preferred_element_type=jnp.float32)
        p = p_ref[r].reshape(CAP, 1)
        y_ref[r, :] = y * p


def _expert_gemms(xbuf, w1, w2, pbuf_col):
    return pl.pallas_call(
        _expert_body,
        grid=(GRID_E // 2,),
        in_specs=[
            pl.BlockSpec((2 * CAP, H), lambda e: (e, 0)),
            pl.BlockSpec((H, 2 * F), lambda e: (0, jnp.minimum(e, E // 2 - 1))),
            pl.BlockSpec((2 * F, H), lambda e: (jnp.minimum(e, E // 2 - 1), 0)),
            pl.BlockSpec((2 * CAP,), lambda e: (e,)),
        ],
        out_specs=pl.BlockSpec((2 * CAP, H), lambda e: (e, 0)),
        out_shape=jax.ShapeDtypeStruct((SLOTS, H), jnp.float32),
    )(xbuf, w1, w2, pbuf_col)


C_PER_W = T // NW
C_CHUNK = 64


@functools.cache
def _combine_kernel():
    @functools.partial(
        pl.kernel,
        out_type=jax.ShapeDtypeStruct((T, H), jnp.float32),
        mesh=_mesh(),
        compiler_params=pltpu.CompilerParams(needs_layout_passes=False),
        scratch_types=[
            pltpu.VMEM((C_CHUNK,), jnp.int32),
            pltpu.VMEM((C_CHUNK, H), jnp.float32),
            pltpu.SemaphoreType.DMA,
        ],
    )
    def _combine(slot_hbm, ybuf_hbm, out_hbm, slot_v, rows_v, sem):
        wid = lax.axis_index("s") * NC + lax.axis_index("c")
        for j in range(C_PER_W // C_CHUNK):
            base = wid * C_PER_W + j * C_CHUNK
            pltpu.sync_copy(slot_hbm.at[pl.ds(base, C_CHUNK)], slot_v)
            pltpu.async_copy(ybuf_hbm.at[slot_v], rows_v, sem).wait()
            pltpu.sync_copy(rows_v, out_hbm.at[pl.ds(base, C_CHUNK)])

    return _combine


def kernel(hidden_states, router_weight, weight1, weight2):
    slot, prob = _router(hidden_states, router_weight)
    idx, pbuf = _build_tables_kernel()(slot, prob)
    xbuf = _dispatch_kernel()(idx, hidden_states)
    ybuf = _expert_gemms(xbuf, weight1, weight2, pbuf)
    return _combine_kernel()(slot, ybuf)


# speedup vs baseline: 2.2060x; 1.0250x over previous
import functools

import jax
import jax.numpy as jnp
from jax import lax
from jax.experimental import pallas as pl
from jax.experimental.pallas import tpu as pltpu
from jax.experimental.pallas import tpu_sc as plsc

E = 64
H = 1024
F = 512
T = 4096
CAP = 128
BT = 512
NB = T // BT
SENT = E * CAP
GRID_E = 66
SLOTS = GRID_E * CAP

NC, NS = 2, 16
NW = NC * NS


def _router_body(x_ref, rw_ref, slot_ref, prob_ref, counts_ref):
    b = pl.program_id(0)

    @pl.when(b == 0)
    def _():
        counts_ref[...] = jnp.zeros_like(counts_ref)

    x = x_ref[...]
    rw = rw_ref[...]
    logits = lax.dot_general(
        rw, x, dimension_numbers=(((0,), (1,)), ((), ())),
        preferred_element_type=jnp.float32)
    m = jnp.max(logits, axis=0, keepdims=True)
    denom = jnp.sum(jnp.exp(logits - m), axis=0)
    prob = 1.0 / denom

    ids = lax.broadcasted_iota(jnp.int32, (E, BT), 0)
    is_max = logits == m
    expert = jnp.min(jnp.where(is_max, ids, E), axis=0)
    onehot = (ids == expert[None, :]).astype(jnp.float32)

    tj = lax.broadcasted_iota(jnp.int32, (BT, BT), 0)
    ti = lax.broadcasted_iota(jnp.int32, (BT, BT), 1)
    tri = (tj < ti).astype(jnp.float32)
    rank_mat = jnp.dot(onehot, tri, preferred_element_type=jnp.float32)
    rank = jnp.sum(rank_mat * onehot, axis=0)
    base = jnp.sum(counts_ref[...] * onehot, axis=0)
    pos = base + rank
    valid = pos < CAP
    slot = jnp.where(valid, expert * CAP + pos.astype(jnp.int32), SENT)
    counts_ref[...] = counts_ref[...] + jnp.sum(onehot, axis=1, keepdims=True)

    slot_ref[...] = slot
    prob_ref[...] = prob


def _router(hidden, rw):
    return pl.pallas_call(
        _router_body,
        grid=(NB,),
        in_specs=[
            pl.BlockSpec((BT, H), lambda b: (b, 0)),
            pl.BlockSpec((H, E), lambda b: (0, 0)),
        ],
        out_specs=[
            pl.BlockSpec((BT,), lambda b: (b,)),
            pl.BlockSpec((BT,), lambda b: (b,)),
        ],
        out_shape=[
            jax.ShapeDtypeStruct((T,), jnp.int32),
            jax.ShapeDtypeStruct((T,), jnp.float32),
        ],
        scratch_shapes=[pltpu.VMEM((E, 1), jnp.float32)],
    )(hidden, rw)


def _mesh():
    return plsc.VectorSubcoreMesh(
        core_axis_name="c", subcore_axis_name="s", num_cores=NC, num_subcores=NS
    )


@functools.cache
def _build_tables_kernel():
    @functools.partial(
        pl.kernel,
        out_type=(
            jax.ShapeDtypeStruct((SLOTS,), jnp.int32),
            jax.ShapeDtypeStruct((SLOTS,), jnp.float32),
        ),
        mesh=_mesh(),
        compiler_params=pltpu.CompilerParams(needs_layout_passes=False),
        scratch_types=[
            pltpu.VMEM((T,), jnp.int32),
            pltpu.VMEM((T,), jnp.float32),
            pltpu.VMEM((SLOTS,), jnp.int32),
            pltpu.VMEM((SLOTS,), jnp.float32),
        ],
    )
    def _build_tables(slot_hbm, prob_hbm, idx_hbm, pbuf_hbm,
                      slot_v, prob_v, idx_v, pbuf_v):
        wid = lax.axis_index("s") * NC + lax.axis_index("c")

        @pl.when(wid == 0)
        def _():
            pltpu.sync_copy(slot_hbm, slot_v)
            pltpu.sync_copy(prob_hbm, prob_v)

            zf = jnp.zeros((16,), jnp.float32)

            def init(i, carry):
                idx_v[pl.ds(i * 16, 16)] = (lax.iota(jnp.int32, 16) + i * 16) & (T - 1)
                pbuf_v[pl.ds(i * 16, 16)] = zf
                return carry

            lax.fori_loop(0, SLOTS // 16, init, 0)

            def body(i, carry):
                s = slot_v[pl.ds(i * 16, 16)]
                p = prob_v[pl.ds(i * 16, 16)]
                t = lax.iota(jnp.int32, 16) + i * 16
                msk = s < SENT
                plsc.store_scatter(idx_v, [s], t, mask=msk)
                plsc.store_scatter(pbuf_v, [s], p, mask=msk)
                return carry

            lax.fori_loop(0, T // 16, body, 0)
            pltpu.sync_copy(idx_v, idx_hbm)
            pltpu.sync_copy(pbuf_v, pbuf_hbm)

    return _build_tables


D_CHUNK = 48
D_NCHUNK = SLOTS // D_CHUNK
D_ROUNDS = (D_NCHUNK + NW - 1) // NW


@functools.cache
def _dispatch_kernel():
    @functools.partial(
        pl.kernel,
        out_type=jax.ShapeDtypeStruct((SLOTS, H), jnp.float32),
        mesh=_mesh(),
        compiler_params=pltpu.CompilerParams(needs_layout_passes=False),
        scratch_types=[
            pltpu.VMEM((D_CHUNK,), jnp.int32),
            pltpu.VMEM((D_CHUNK, H), jnp.float32),
            pltpu.VMEM((D_CHUNK, H), jnp.float32),
            pltpu.SemaphoreType.DMA,
            pltpu.SemaphoreType.DMA,
            pltpu.SemaphoreType.DMA,
            pltpu.SemaphoreType.DMA,
        ],
    )
    def _dispatch(idx_hbm, hid_hbm, xbuf_hbm, idx_v, rows_a, rows_b,
                  gsem_a, gsem_b, wsem_a, wsem_b):
        wid = lax.axis_index("s") * NC + lax.axis_index("c")
        rows = (rows_a, rows_b)
        gsem = (gsem_a, gsem_b)
        wsem = (wsem_a, wsem_b)

        def cid(j):
            return j * NW + wid

        def valid(j):
            return cid(j) < D_NCHUNK

        def start_gather(j):
            pltpu.sync_copy(idx_hbm.at[pl.ds(cid(j) * D_CHUNK, D_CHUNK)], idx_v)
            pltpu.make_async_copy(
                hid_hbm.at[idx_v], rows[j & 1], gsem[j & 1]).start()

        def wait_gather(j):
            pltpu.make_async_copy(
                hid_hbm.at[idx_v], rows[j & 1], gsem[j & 1]).wait()

        def start_wb(j):
            pltpu.make_async_copy(
                rows[j & 1], xbuf_hbm.at[pl.ds(cid(j) * D_CHUNK, D_CHUNK)],
                wsem[j & 1]).start()

        def wait_wb(j):
            pltpu.make_async_copy(
                rows[j & 1], xbuf_hbm.at[pl.ds(cid(j) * D_CHUNK, D_CHUNK)],
                wsem[j & 1]).wait()

        pl.when(valid(0))(lambda: start_gather(0))
        for j in range(D_ROUNDS):
            pl.when(valid(j))(lambda j=j: wait_gather(j))
            if j + 1 < D_ROUNDS:
                def _pref(j=j):
                    if j + 1 >= 2:
                        wait_wb(j - 1)
                    start_gather(j + 1)
                pl.when(valid(j + 1))(_pref)
            pl.when(valid(j))(lambda j=j: start_wb(j))
        @pl.when(valid(D_ROUNDS - 1))
        def _():
            wait_wb(D_ROUNDS - 1)
            wait_wb(D_ROUNDS - 2)

        @pl.when(jnp.logical_not(valid(D_ROUNDS - 1)))
        def _():
            wait_wb(D_ROUNDS - 2)
            wait_wb(D_ROUNDS - 3)

    return _dispatch


def _expert_body(x_ref, w1_ref, w2_ref, p_ref, y_ref):
    for k in range(2):
        r = pl.ds(k * CAP, CAP)
        c = pl.ds(k * F, F)
        x = x_ref[r, :]
        h = jnp.dot(x, w1_ref[:, c], preferred_element_type=jnp.float32)
        a = h * lax.logistic(h)
        y = jnp.dot(a, w2_ref[c, :], preferred_element_type=jnp.float32)
        p = p_ref[r].reshape(CAP, 1)
        y_ref[r, :] = y * p


def _expert_gemms(xbuf, w1, w2, pbuf_col):
    return pl.pallas_call(
        _expert_body,
        grid=(GRID_E // 2,),
        in_specs=[
            pl.BlockSpec((2 * CAP, H), lambda e: (e, 0)),
            pl.BlockSpec((H, 2 * F), lambda e: (0, jnp.minimum(e, E // 2 - 1))),
            pl.BlockSpec((2 * F, H), lambda e: (jnp.minimum(e, E // 2 - 1), 0)),
            pl.BlockSpec((2 * CAP,), lambda e: (e,)),
        ],
        out_specs=pl.BlockSpec((2 * CAP, H), lambda e: (e, 0)),
        out_shape=jax.ShapeDtypeStruct((SLOTS, H), jnp.float32),
    )(xbuf, w1, w2, pbuf_col)


C_PER_W = T // NW
C_CHUNK = 64


@functools.cache
def _combine_kernel():
    @functools.partial(
        pl.kernel,
        out_type=jax.ShapeDtypeStruct((T, H), jnp.float32),
        mesh=_mesh(),
        compiler_params=pltpu.CompilerParams(needs_layout_passes=False),
        scratch_types=[
            pltpu.VMEM((C_CHUNK,), jnp.int32),
            pltpu.VMEM((C_CHUNK, H), jnp.float32),
            pltpu.SemaphoreType.DMA,
        ],
    )
    def _combine(slot_hbm, ybuf_hbm, out_hbm, slot_v, rows_v, sem):
        wid = lax.axis_index("s") * NC + lax.axis_index("c")
        for j in range(C_PER_W // C_CHUNK):
            base = wid * C_PER_W + j * C_CHUNK
            pltpu.sync_copy(slot_hbm.at[pl.ds(base, C_CHUNK)], slot_v)
            pltpu.async_copy(ybuf_hbm.at[slot_v], rows_v, sem).wait()
            pltpu.sync_copy(rows_v, out_hbm.at[pl.ds(base, C_CHUNK)])

    return _combine


def kernel(hidden_states, router_weight, weight1, weight2):
    slot, prob = _router(hidden_states, router_weight)
    idx, pbuf = _build_tables_kernel()(slot, prob)
    xbuf = _dispatch_kernel()(idx, hidden_states)
    ybuf = _expert_gemms(xbuf, weight1, weight2, pbuf)
    return _combine_kernel()(slot, ybuf)


# speedup vs baseline: 2.2090x; 1.0014x over previous
import functools

import jax
import jax.numpy as jnp
from jax import lax
from jax.experimental import pallas as pl
from jax.experimental.pallas import tpu as pltpu
from jax.experimental.pallas import tpu_sc as plsc

E = 64
H = 1024
F = 512
T = 4096
CAP = 128
BT = 512
NB = T // BT
SENT = E * CAP
GRID_E = 66
SLOTS = GRID_E * CAP

NC, NS = 2, 16
NW = NC * NS


def _router_body(x_ref, rw_ref, slot_ref, prob_ref, counts_ref):
    b = pl.program_id(0)

    @pl.when(b == 0)
    def _():
        counts_ref[...] = jnp.zeros_like(counts_ref)

    x = x_ref[...]
    rw = rw_ref[...]
    logits = lax.dot_general(
        rw, x, dimension_numbers=(((0,), (1,)), ((), ())),
        preferred_element_type=jnp.float32)
    m = jnp.max(logits, axis=0, keepdims=True)
    denom = jnp.sum(jnp.exp(logits - m), axis=0)
    prob = 1.0 / denom

    ids = lax.broadcasted_iota(jnp.int32, (E, BT), 0)
    is_max = logits == m
    expert = jnp.min(jnp.where(is_max, ids, E), axis=0)
    onehot = (ids == expert[None, :]).astype(jnp.float32)

    tj = lax.broadcasted_iota(jnp.int32, (BT, BT), 0)
    ti = lax.broadcasted_iota(jnp.int32, (BT, BT), 1)
    tri = (tj < ti).astype(jnp.float32)
    rank_mat = jnp.dot(onehot, tri, preferred_element_type=jnp.float32)
    rank = jnp.sum(rank_mat * onehot, axis=0)
    base = jnp.sum(counts_ref[...] * onehot, axis=0)
    pos = base + rank
    valid = pos < CAP
    slot = jnp.where(valid, expert * CAP + pos.astype(jnp.int32), SENT)
    counts_ref[...] = counts_ref[...] + jnp.sum(onehot, axis=1, keepdims=True)

    slot_ref[...] = slot
    prob_ref[...] = prob


def _router(hidden, rw):
    return pl.pallas_call(
        _router_body,
        grid=(NB,),
        in_specs=[
            pl.BlockSpec((BT, H), lambda b: (b, 0)),
            pl.BlockSpec((H, E), lambda b: (0, 0)),
        ],
        out_specs=[
            pl.BlockSpec((BT,), lambda b: (b,)),
            pl.BlockSpec((BT,), lambda b: (b,)),
        ],
        out_shape=[
            jax.ShapeDtypeStruct((T,), jnp.int32),
            jax.ShapeDtypeStruct((T,), jnp.float32),
        ],
        scratch_shapes=[pltpu.VMEM((E, 1), jnp.float32)],
    )(hidden, rw)


def _mesh():
    return plsc.VectorSubcoreMesh(
        core_axis_name="c", subcore_axis_name="s", num_cores=NC, num_subcores=NS
    )


@functools.cache
def _build_tables_kernel():
    @functools.partial(
        pl.kernel,
        out_type=(
            jax.ShapeDtypeStruct((SLOTS,), jnp.int32),
            jax.ShapeDtypeStruct((SLOTS,), jnp.float32),
        ),
        mesh=_mesh(),
        compiler_params=pltpu.CompilerParams(needs_layout_passes=False),
        scratch_types=[
            pltpu.VMEM((T,), jnp.int32),
            pltpu.VMEM((T,), jnp.float32),
            pltpu.VMEM((SLOTS,), jnp.int32),
            pltpu.VMEM((SLOTS,), jnp.float32),
        ],
    )
    def _build_tables(slot_hbm, prob_hbm, idx_hbm, pbuf_hbm,
                      slot_v, prob_v, idx_v, pbuf_v):
        wid = lax.axis_index("s") * NC + lax.axis_index("c")

        @pl.when(wid == 0)
        def _():
            pltpu.sync_copy(slot_hbm, slot_v)
            pltpu.sync_copy(prob_hbm, prob_v)

            zf = jnp.zeros((16,), jnp.float32)

            def init(i, carry):
                idx_v[pl.ds(i * 16, 16)] = (lax.iota(jnp.int32, 16) + i * 16) & (T - 1)
                pbuf_v[pl.ds(i * 16, 16)] = zf
                return carry

            lax.fori_loop(0, SLOTS // 16, init, 0)

            def body(i, carry):
                s = slot_v[pl.ds(i * 16, 16)]
                p = prob_v[pl.ds(i * 16, 16)]
                t = lax.iota(jnp.int32, 16) + i * 16
                msk = s < SENT
                plsc.store_scatter(idx_v, [s], t, mask=msk)
                plsc.store_scatter(pbuf_v, [s], p, mask=msk)
                return carry

            lax.fori_loop(0, T // 16, body, 0)
            pltpu.sync_copy(idx_v, idx_hbm)
            pltpu.sync_copy(pbuf_v, pbuf_hbm)

    return _build_tables


D_CHUNK = 48
D_NCHUNK = SLOTS // D_CHUNK
D_ROUNDS = (D_NCHUNK + NW - 1) // NW


@functools.cache
def _dispatch_kernel():
    @functools.partial(
        pl.kernel,
        out_type=jax.ShapeDtypeStruct((SLOTS, H), jnp.float32),
        mesh=_mesh(),
        compiler_params=pltpu.CompilerParams(needs_layout_passes=False),
        scratch_types=[
            pltpu.VMEM((D_CHUNK,), jnp.int32),
            pltpu.VMEM((D_CHUNK, H), jnp.float32),
            pltpu.VMEM((D_CHUNK, H), jnp.float32),
            pltpu.SemaphoreType.DMA,
            pltpu.SemaphoreType.DMA,
            pltpu.SemaphoreType.DMA,
            pltpu.SemaphoreType.DMA,
        ],
    )
    def _dispatch(idx_hbm, hid_hbm, xbuf_hbm, idx_v, rows_a, rows_b,
                  gsem_a, gsem_b, wsem_a, wsem_b):
        wid = lax.axis_index("s") * NC + lax.axis_index("c")
        rows = (rows_a, rows_b)
        gsem = (gsem_a, gsem_b)
        wsem = (wsem_a, wsem_b)

        def cid(j):
            return j * NW + wid

        def valid(j):
            return cid(j) < D_NCHUNK

        def start_gather(j):
            pltpu.sync_copy(idx_hbm.at[pl.ds(cid(j) * D_CHUNK, D_CHUNK)], idx_v)
            pltpu.make_async_copy(
                hid_hbm.at[idx_v], rows[j & 1], gsem[j & 1]).start()

        def wait_gather(j):
            pltpu.make_async_copy(
                hid_hbm.at[idx_v], rows[j & 1], gsem[j & 1]).wait()

        def start_wb(j):
            pltpu.make_async_copy(
                rows[j & 1], xbuf_hbm.at[pl.ds(cid(j) * D_CHUNK, D_CHUNK)],
                wsem[j & 1]).start()

        def wait_wb(j):
            pltpu.make_async_copy(
                rows[j & 1], xbuf_hbm.at[pl.ds(cid(j) * D_CHUNK, D_CHUNK)],
                wsem[j & 1]).wait()

        pl.when(valid(0))(lambda: start_gather(0))
        for j in range(D_ROUNDS):
            pl.when(valid(j))(lambda j=j: wait_gather(j))
            if j + 1 < D_ROUNDS:
                def _pref(j=j):
                    if j + 1 >= 2:
                        wait_wb(j - 1)
                    start_gather(j + 1)
                pl.when(valid(j + 1))(_pref)
            pl.when(valid(j))(lambda j=j: start_wb(j))
        @pl.when(valid(D_ROUNDS - 1))
        def _():
            wait_wb(D_ROUNDS - 1)
            wait_wb(D_ROUNDS - 2)

        @pl.when(jnp.logical_not(valid(D_ROUNDS - 1)))
        def _():
            wait_wb(D_ROUNDS - 2)
            wait_wb(D_ROUNDS - 3)

    return _dispatch


EPB = 4


def _expert_body(x_ref, w1_ref, w2_ref, p_ref, y_ref):
    for k in range(EPB):
        r = pl.ds(k * CAP, CAP)
        c = pl.ds(k * F, F)
        x = x_ref[r, :]
        h = jnp.dot(x, w1_ref[:, c], preferred_element_type=jnp.float32)
        a = h * lax.logistic(h)
        y = jnp.dot(a, w2_ref[c, :], preferred_element_type=jnp.float32)
        p = p_ref[r].reshape(CAP, 1)
        y_ref[r, :] = y * p


def _expert_gemms(xbuf, w1, w2, pbuf_col):
    return pl.pallas_call(
        _expert_body,
        grid=(GRID_E // EPB,),
        in_specs=[
            pl.BlockSpec((EPB * CAP, H), lambda e: (e, 0)),
            pl.BlockSpec((H, EPB * F), lambda e: (0, jnp.minimum(e, E // EPB - 1))),
            pl.BlockSpec((EPB * F, H), lambda e: (jnp.minimum(e, E // EPB - 1), 0)),
            pl.BlockSpec((EPB * CAP,), lambda e: (e,)),
        ],
        out_specs=pl.BlockSpec((EPB * CAP, H), lambda e: (e, 0)),
        out_shape=jax.ShapeDtypeStruct((SLOTS, H), jnp.float32),
    )(xbuf, w1, w2, pbuf_col)


C_PER_W = T // NW
C_CHUNK = 64


@functools.cache
def _combine_kernel():
    @functools.partial(
        pl.kernel,
        out_type=jax.ShapeDtypeStruct((T, H), jnp.float32),
        mesh=_mesh(),
        compiler_params=pltpu.CompilerParams(needs_layout_passes=False),
        scratch_types=[
            pltpu.VMEM((C_CHUNK,), jnp.int32),
            pltpu.VMEM((C_CHUNK, H), jnp.float32),
            pltpu.SemaphoreType.DMA,
        ],
    )
    def _combine(slot_hbm, ybuf_hbm, out_hbm, slot_v, rows_v, sem):
        wid = lax.axis_index("s") * NC + lax.axis_index("c")
        for j in range(C_PER_W // C_CHUNK):
            base = wid * C_PER_W + j * C_CHUNK
            pltpu.sync_copy(slot_hbm.at[pl.ds(base, C_CHUNK)], slot_v)
            pltpu.async_copy(ybuf_hbm.at[slot_v], rows_v, sem).wait()
            pltpu.sync_copy(rows_v, out_hbm.at[pl.ds(base, C_CHUNK)])

    return _combine


def kernel(hidden_states, router_weight, weight1, weight2):
    slot, prob = _router(hidden_states, router_weight)
    idx, pbuf = _build_tables_kernel()(slot, prob)
    xbuf = _dispatch_kernel()(idx, hidden_states)
    ybuf = _expert_gemms(xbuf, weight1, weight2, pbuf)
    return _combine_kernel()(slot, ybuf)
